# baseline (speedup 1.0000x reference)
import numpy as np
import jax
import jax.numpy as jnp
from jax.experimental import pallas as pl
from jax.experimental.pallas import tpu as pltpu

N_ROWS = 100000
MUL = 128
INV_SQRT3_ = 0.5773502691896258
BLOCK = 1000

_T1 = np.tile(np.eye(3, dtype=np.float32), (1, 128))


def _body(x0_ref, x1_ref, y0_ref, y1_ref, t1_ref, w0_ref, wsv_ref, wvs_ref,
          b_ref, o_ref):
    bf16 = jnp.bfloat16
    f32 = jnp.float32
    ones_row = jnp.ones((1, MUL * 3), dtype=bf16)
    y0_384 = jnp.dot(y0_ref[...].astype(bf16), ones_row,
                     preferred_element_type=f32)
    y_tile = jnp.dot(y1_ref[...].astype(bf16), t1_ref[...],
                     preferred_element_type=f32)

    x0 = x0_ref[...]
    x1 = x1_ref[...]

    seg_ss = x0 * y0_384[:, :128]
    p = x1 * y_tile
    l0 = jnp.concatenate([seg_ss, p], axis=1).astype(bf16)
    out0 = jnp.dot(l0, w0_ref[...], preferred_element_type=f32) + b_ref[...]

    sv = jnp.dot(x0.astype(bf16), wsv_ref[...], preferred_element_type=f32)
    vs = jnp.dot(x1.astype(bf16), wvs_ref[...], preferred_element_type=f32)
    o_ref[:, :128] = out0
    o_ref[:, 128:] = sv * y_tile + vs * y0_384


def kernel(x_0e, x_1o, y_0e, y_1o, w_ss, w_vv, w_sv, w_vs, b):
    n = x_0e.shape[0]
    x1f = x_1o.reshape(n, MUL * 3)

    bf16 = jnp.bfloat16
    t1_bf16 = jnp.asarray(_T1, dtype=bf16)
    w0 = jnp.concatenate(
        [w_ss, jnp.repeat(w_vv * INV_SQRT3_, 3, axis=0)], axis=0
    ).astype(bf16)
    wsv = jnp.repeat(w_sv, 3, axis=1).astype(bf16)
    wvs = (w_vs[:, None, :, None] * jnp.eye(3, dtype=w_vs.dtype)[None, :, None, :]
           ).reshape(MUL * 3, MUL * 3).astype(bf16)
    b2 = b.reshape(1, MUL)

    grid = n // BLOCK
    row_spec = lambda width: pl.BlockSpec((BLOCK, width), lambda i: (i, 0))
    full_spec = lambda a: pl.BlockSpec(a.shape, lambda i: (0, 0))

    return pl.pallas_call(
        _body,
        grid=(grid,),
        in_specs=[
            row_spec(MUL),
            row_spec(MUL * 3),
            row_spec(1),
            row_spec(3),
            full_spec(t1_bf16),
            full_spec(w0),
            full_spec(wsv),
            full_spec(wvs),
            full_spec(b2),
        ],
        out_specs=row_spec(MUL * 4),
        out_shape=jax.ShapeDtypeStruct((n, MUL * 4), jnp.float32),
        compiler_params=pltpu.CompilerParams(
            dimension_semantics=("parallel",),
            vmem_limit_bytes=50 * 1024 * 1024,
        ),
    )(x_0e, x1f, y_0e, y_1o, t1_bf16, w0, wsv, wvs, b2)


# speedup vs baseline: 1.1287x; 1.1287x over previous
import numpy as np
import jax
import jax.numpy as jnp
from jax.experimental import pallas as pl
from jax.experimental.pallas import tpu as pltpu

N_ROWS = 100000
MUL = 128
INV_SQRT3_ = 0.5773502691896258
BLOCK = 2000

_T1 = np.tile(np.eye(3, dtype=np.float32), (1, 128))


def _body(x0_ref, x1_ref, t1_ref, w0_ref, wsv_ref, wvs_ref,
          b_ref, o_ref):
    bf16 = jnp.bfloat16
    f32 = jnp.float32
    ones_row = jnp.ones((1, MUL * 3), dtype=bf16)
    y0_384 = jnp.dot(x0_ref[:, 0:1].astype(bf16), ones_row,
                     preferred_element_type=f32)
    y_tile = jnp.dot(x0_ref[:, 1:4].astype(bf16), t1_ref[...],
                     preferred_element_type=f32)

    x0 = x0_ref[...]
    x1 = x1_ref[...]

    seg_ss = x0 * y0_384[:, :128]
    p = x1 * y_tile
    l0 = jnp.concatenate([seg_ss, p], axis=1).astype(bf16)
    out0 = jnp.dot(l0, w0_ref[...], preferred_element_type=f32) + b_ref[...]

    sv = jnp.dot(x0.astype(bf16), wsv_ref[...], preferred_element_type=f32)
    vs = jnp.dot(x1.astype(bf16), wvs_ref[...], preferred_element_type=f32)
    o_ref[:, :128] = out0
    o_ref[:, 128:] = sv * y_tile + vs * y0_384


def kernel(x_0e, x_1o, y_0e, y_1o, w_ss, w_vv, w_sv, w_vs, b):
    n = x_0e.shape[0]
    x1f = x_1o.reshape(n, MUL * 3)

    bf16 = jnp.bfloat16
    t1_bf16 = jnp.asarray(_T1, dtype=bf16)
    w0 = jnp.concatenate(
        [w_ss, jnp.repeat(w_vv * INV_SQRT3_, 3, axis=0)], axis=0
    ).astype(bf16)
    wsv = jnp.repeat(w_sv, 3, axis=1).astype(bf16)
    wvs = (w_vs[:, None, :, None] * jnp.eye(3, dtype=w_vs.dtype)[None, :, None, :]
           ).reshape(MUL * 3, MUL * 3).astype(bf16)
    b2 = b.reshape(1, MUL)

    grid = n // BLOCK
    row_spec = lambda width: pl.BlockSpec((BLOCK, width), lambda i: (i, 0))
    full_spec = lambda a: pl.BlockSpec(a.shape, lambda i: (0, 0))

    return pl.pallas_call(
        _body,
        grid=(grid,),
        in_specs=[
            row_spec(MUL),
            row_spec(MUL * 3),
            full_spec(t1_bf16),
            full_spec(w0),
            full_spec(wsv),
            full_spec(wvs),
            full_spec(b2),
        ],
        out_specs=row_spec(MUL * 4),
        out_shape=jax.ShapeDtypeStruct((n, MUL * 4), jnp.float32),
        compiler_params=pltpu.CompilerParams(
            dimension_semantics=("arbitrary",),
            vmem_limit_bytes=50 * 1024 * 1024,
        ),
    )(x_0e, x1f, t1_bf16, w0, wsv, wvs, b2)


# speedup vs baseline: 3.5467x; 3.1423x over previous
import numpy as np
import jax
import jax.numpy as jnp
from jax.experimental import pallas as pl
from jax.experimental.pallas import tpu as pltpu

MUL = 128
INV_SQRT3_ = 0.5773502691896258
BLOCK = 2000

_T4 = np.kron(np.eye(4, dtype=np.float32), np.ones((1, MUL), np.float32))


def _body(x0_ref, x1_ref, m_ref, t4_ref, w0_ref, w1_ref,
          b_ref, o_ref):
    bf16 = jnp.bfloat16
    f32 = jnp.float32
    yy = jnp.dot(m_ref[...].astype(bf16), t4_ref[...],
                 preferred_element_type=f32)
    y0 = yy[:, :MUL]
    y1 = [yy[:, MUL:2 * MUL], yy[:, 2 * MUL:3 * MUL], yy[:, 3 * MUL:]]

    x0 = x0_ref[...]
    x1 = [x1_ref[0], x1_ref[1], x1_ref[2]]

    l0 = jnp.concatenate(
        [x0 * y0, x1[0] * y1[0], x1[1] * y1[1], x1[2] * y1[2]], axis=1
    ).astype(bf16)
    o_ref[:, :MUL] = (
        jnp.dot(l0, w0_ref[...], preferred_element_type=f32) + b_ref[...]
    )

    l1 = jnp.concatenate(
        [x0 * y1[0], x0 * y1[1], x0 * y1[2],
         y0 * x1[0], y0 * x1[1], y0 * x1[2]], axis=1
    ).astype(bf16)
    o_ref[:, MUL:] = jnp.dot(l1, w1_ref[...], preferred_element_type=f32)


def _perm_kron(w):
    eye3 = jnp.eye(3, dtype=w.dtype)
    return jnp.einsum("ij,uw->iuwj", eye3, w).reshape(3 * MUL, 3 * MUL)


def kernel(x_0e, x_1o, y_0e, y_1o, w_ss, w_vv, w_sv, w_vs, b):
    n = x_0e.shape[0]
    x1t = jnp.transpose(x_1o, (2, 0, 1))
    m = jnp.concatenate([y_0e, y_1o], axis=1)

    bf16 = jnp.bfloat16
    t4 = jnp.asarray(_T4, dtype=bf16)
    wvv = w_vv * INV_SQRT3_
    w0 = jnp.concatenate([w_ss, wvv, wvv, wvv], axis=0).astype(bf16)
    w1 = jnp.concatenate(
        [_perm_kron(w_sv), _perm_kron(w_vs)], axis=0
    ).astype(bf16)
    b2 = b.reshape(1, MUL)

    grid = n // BLOCK
    row_spec = lambda width: pl.BlockSpec((BLOCK, width), lambda i: (i, 0))
    full_spec = lambda a: pl.BlockSpec(a.shape, lambda i: (0, 0))

    return pl.pallas_call(
        _body,
        grid=(grid,),
        in_specs=[
            row_spec(MUL),
            pl.BlockSpec((3, BLOCK, MUL), lambda i: (0, i, 0)),
            row_spec(4),
            full_spec(t4),
            full_spec(w0),
            full_spec(w1),
            full_spec(b2),
        ],
        out_specs=row_spec(MUL * 4),
        out_shape=jax.ShapeDtypeStruct((n, MUL * 4), jnp.float32),
        compiler_params=pltpu.CompilerParams(
            dimension_semantics=("arbitrary",),
            vmem_limit_bytes=50 * 1024 * 1024,
        ),
    )(x_0e, x1t, m, t4, w0, w1, b2)


# speedup vs baseline: 4.0551x; 1.1434x over previous
import numpy as np
import jax
import jax.numpy as jnp
from jax.experimental import pallas as pl
from jax.experimental.pallas import tpu as pltpu

MUL = 128
INV_SQRT3_ = 0.5773502691896258
BLOCK = 2000

_T4 = np.kron(np.eye(4, dtype=np.float32), np.ones((1, MUL), np.float32))


def _body(x0_ref, x1_ref, yt_ref, t4_ref, w0_ref, w1_ref,
          b_ref, o_ref):
    bf16 = jnp.bfloat16
    f32 = jnp.float32
    yy = jax.lax.dot_general(
        yt_ref[0].astype(bf16), t4_ref[...],
        (((0,), (0,)), ((), ())),
        preferred_element_type=f32)
    y0 = yy[:, :MUL]
    y1 = [yy[:, MUL:2 * MUL], yy[:, 2 * MUL:3 * MUL], yy[:, 3 * MUL:]]

    x0 = x0_ref[...]
    x1 = [x1_ref[0], x1_ref[1], x1_ref[2]]

    l0 = jnp.concatenate(
        [x0 * y0, x1[0] * y1[0], x1[1] * y1[1], x1[2] * y1[2]], axis=1
    ).astype(bf16)
    o_ref[:, :MUL] = (
        jnp.dot(l0, w0_ref[...], preferred_element_type=f32) + b_ref[...]
    )

    l1 = jnp.concatenate(
        [x0 * y1[0], x0 * y1[1], x0 * y1[2],
         y0 * x1[0], y0 * x1[1], y0 * x1[2]], axis=1
    ).astype(bf16)
    o_ref[:, MUL:] = jnp.dot(l1, w1_ref[...], preferred_element_type=f32)


def _perm_kron(w):
    eye3 = jnp.eye(3, dtype=w.dtype)
    return jnp.einsum("ij,uw->iuwj", eye3, w).reshape(3 * MUL, 3 * MUL)


def kernel(x_0e, x_1o, y_0e, y_1o, w_ss, w_vv, w_sv, w_vs, b):
    n = x_0e.shape[0]
    x1t = jnp.transpose(x_1o, (2, 0, 1))
    yt = jnp.concatenate([y_0e.T, y_1o.T], axis=0)
    ytr = yt.reshape(4, n // BLOCK, BLOCK).transpose(1, 0, 2)

    bf16 = jnp.bfloat16
    t4 = jnp.asarray(_T4, dtype=bf16)
    wvv = w_vv * INV_SQRT3_
    w0 = jnp.concatenate([w_ss, wvv, wvv, wvv], axis=0).astype(bf16)
    w1 = jnp.concatenate(
        [_perm_kron(w_sv), _perm_kron(w_vs)], axis=0
    ).astype(bf16)
    b2 = b.reshape(1, MUL)

    grid = n // BLOCK
    row_spec = lambda width: pl.BlockSpec((BLOCK, width), lambda i: (i, 0))
    full_spec = lambda a: pl.BlockSpec(a.shape, lambda i: (0, 0))

    return pl.pallas_call(
        _body,
        grid=(grid,),
        in_specs=[
            row_spec(MUL),
            pl.BlockSpec((3, BLOCK, MUL), lambda i: (0, i, 0)),
            pl.BlockSpec((1, 4, BLOCK), lambda i: (i, 0, 0)),
            full_spec(t4),
            full_spec(w0),
            full_spec(w1),
            full_spec(b2),
        ],
        out_specs=row_spec(MUL * 4),
        out_shape=jax.ShapeDtypeStruct((n, MUL * 4), jnp.float32),
        compiler_params=pltpu.CompilerParams(
            dimension_semantics=("arbitrary",),
            vmem_limit_bytes=50 * 1024 * 1024,
        ),
    )(x_0e, x1t, ytr, t4, w0, w1, b2)


# speedup vs baseline: 4.0693x; 1.0035x over previous
import numpy as np
import jax
import jax.numpy as jnp
from jax.experimental import pallas as pl
from jax.experimental.pallas import tpu as pltpu

MUL = 128
INV_SQRT3_ = 0.5773502691896258
BLOCK = 2000

_T4 = np.kron(np.eye(4, dtype=np.float32), np.ones((1, MUL), np.float32))


def _body(x0_ref, x1_ref, yt_ref, t4_ref, w0_ref, w1_ref,
          b_ref, o_ref):
    bf16 = jnp.bfloat16
    f32 = jnp.float32
    yy = jax.lax.dot_general(
        yt_ref[0].astype(bf16), t4_ref[...],
        (((0,), (0,)), ((), ())),
        preferred_element_type=f32).astype(bf16)
    y0 = yy[:, :MUL]
    y1 = [yy[:, MUL:2 * MUL], yy[:, 2 * MUL:3 * MUL], yy[:, 3 * MUL:]]

    x0 = x0_ref[...].astype(bf16)
    x1 = [x1_ref[0].astype(bf16), x1_ref[1].astype(bf16),
          x1_ref[2].astype(bf16)]

    l0 = jnp.concatenate(
        [x0 * y0, x1[0] * y1[0], x1[1] * y1[1], x1[2] * y1[2]], axis=1
    )
    o_ref[:, :MUL] = (
        jnp.dot(l0, w0_ref[...], preferred_element_type=f32) + b_ref[...]
    )

    l1 = jnp.concatenate(
        [x0 * y1[0], x0 * y1[1], x0 * y1[2],
         y0 * x1[0], y0 * x1[1], y0 * x1[2]], axis=1
    )
    o_ref[:, MUL:] = jnp.dot(l1, w1_ref[...], preferred_element_type=f32)


def _perm_kron(w):
    eye3 = jnp.eye(3, dtype=w.dtype)
    return jnp.einsum("ij,uw->iuwj", eye3, w).reshape(3 * MUL, 3 * MUL)


def kernel(x_0e, x_1o, y_0e, y_1o, w_ss, w_vv, w_sv, w_vs, b):
    n = x_0e.shape[0]
    x1t = jnp.transpose(x_1o, (2, 0, 1))
    yt = jnp.concatenate([y_0e.T, y_1o.T], axis=0)
    ytr = yt.reshape(4, n // BLOCK, BLOCK).transpose(1, 0, 2)

    bf16 = jnp.bfloat16
    t4 = jnp.asarray(_T4, dtype=bf16)
    wvv = w_vv * INV_SQRT3_
    w0 = jnp.concatenate([w_ss, wvv, wvv, wvv], axis=0).astype(bf16)
    w1 = jnp.concatenate(
        [_perm_kron(w_sv), _perm_kron(w_vs)], axis=0
    ).astype(bf16)
    b2 = b.reshape(1, MUL)

    grid = n // BLOCK
    row_spec = lambda width: pl.BlockSpec((BLOCK, width), lambda i: (i, 0))
    full_spec = lambda a: pl.BlockSpec(a.shape, lambda i: (0, 0))

    return pl.pallas_call(
        _body,
        grid=(grid,),
        in_specs=[
            row_spec(MUL),
            pl.BlockSpec((3, BLOCK, MUL), lambda i: (0, i, 0)),
            pl.BlockSpec((1, 4, BLOCK), lambda i: (i, 0, 0)),
            full_spec(t4),
            full_spec(w0),
            full_spec(w1),
            full_spec(b2),
        ],
        out_specs=row_spec(MUL * 4),
        out_shape=jax.ShapeDtypeStruct((n, MUL * 4), jnp.float32),
        compiler_params=pltpu.CompilerParams(
            dimension_semantics=("arbitrary",),
            vmem_limit_bytes=50 * 1024 * 1024,
        ),
    )(x_0e, x1t, ytr, t4, w0, w1, b2)
